# Initial kernel scaffold; baseline (speedup 1.0000x reference)
import jax
import jax.numpy as jnp
from jax.experimental import pallas as pl


def kernel(sensor_data, sensor_mask):
    raise NotImplementedError("write your pallas kernel here")


# speedup vs baseline: 45.7456x; 45.7456x over previous
import functools

import jax
import jax.numpy as jnp
from jax import lax
from jax.experimental import pallas as pl
from jax.experimental.pallas import tpu as pltpu
from jax.experimental.pallas import tpu_sc as plsc

_NX = 256
_NY = 256
_DX = 0.001
_DY = 0.001
_VS = 1550.0
_DT = 5e-08
_S = 128
_T = 5000
_P = _NX * _NY

_KS = 8
_NCHUNK = _S // _KS
_PC = 2048
_NPC = _P // _PC
_NV = _PC // 16


def _delay_body(mask_ref, t_ref):
    pcb = pl.program_id(0)
    pi = (lax.broadcasted_iota(jnp.int32, (_S, _PC), 1) + pcb * _PC)
    ix = (pi // _NY).astype(jnp.float32)
    iy = (pi % _NY).astype(jnp.float32)
    x = mask_ref[:, 0].astype(jnp.float32)[:, None]
    y = mask_ref[:, 1].astype(jnp.float32)[:, None]
    ddx = (x - ix) * _DX
    ddy = (y - iy) * _DY
    dis = jnp.sqrt(ddx * ddx + ddy * ddy)
    t_ref[...] = (dis / _VS / _DT).astype(jnp.int32).reshape(1, 1, _S * _PC)


def _delays(sensor_mask):
    return pl.pallas_call(
        _delay_body,
        grid=(_NPC,),
        in_specs=[pl.BlockSpec((_S, 2), lambda i: (0, 0))],
        out_specs=pl.BlockSpec((1, 1, _S * _PC), lambda i: (i, 0, 0)),
        out_shape=jax.ShapeDtypeStruct((_NPC, 1, _S * _PC), jnp.int32),
    )(sensor_mask).reshape(_NPC, _S * _PC)


def _sc_das(data_hbm, t_hbm, out_hbm, acc, traces, idx):
    w = lax.axis_index("s") * 2 + lax.axis_index("c")

    def zero_body(i, _):
        acc[pl.ds(i * 16, 16)] = jnp.zeros((16,), jnp.float32)
        return _

    lax.fori_loop(0, _P // 16, zero_body, None)

    def chunk_body(jc, _):
        pltpu.sync_copy(data_hbm.at[w, pl.ds(jc * _KS * _T, _KS * _T)], traces)

        def pc_body(pcb, _):
            pltpu.sync_copy(
                t_hbm.at[pcb, pl.ds(jc * _KS * _PC, _KS * _PC)], idx)

            def v_body(v, _):
                sl = pl.ds(pcb * _PC + v * 16, 16)
                a = acc[sl]
                for s in range(_KS):
                    ti = idx[pl.ds(s * _PC + v * 16, 16)]
                    a = a + plsc.load_gather(traces, [ti + (s * _T)])
                acc[sl] = a
                return _

            lax.fori_loop(0, _NV, v_body, None)
            return _

        lax.fori_loop(0, _NPC, pc_body, None)
        return _

    lax.fori_loop(0, _NCHUNK, chunk_body, None)

    def mm_body(i, carry):
        mn, mx = carry
        a = acc[pl.ds(i * 16, 16)]
        return jnp.minimum(mn, a), jnp.maximum(mx, a)

    mn0 = acc[pl.ds(0, 16)]
    mn, mx = lax.fori_loop(0, _P // 16, mm_body, (mn0, mn0))
    mnv = jnp.min(mn)
    mn_vec = jnp.full((16,), mnv, jnp.float32)
    den_vec = jnp.full((16,), jnp.max(mx) - mnv, jnp.float32)

    def norm_body(i, _):
        sl = pl.ds(i * 16, 16)
        acc[sl] = (acc[sl] - mn_vec) / den_vec
        return _

    lax.fori_loop(0, _P // 16, norm_body, None)
    pltpu.sync_copy(acc, out_hbm.at[w])


def kernel(sensor_data, sensor_mask):
    batch = sensor_data.shape[0]
    t = _delays(sensor_mask)
    data = sensor_data.reshape(batch * 2, _S * _T)

    mesh = plsc.VectorSubcoreMesh(core_axis_name="c", subcore_axis_name="s")
    image = pl.kernel(
        _sc_das,
        mesh=mesh,
        compiler_params=pltpu.CompilerParams(
            use_tc_tiling_on_sc=False, needs_layout_passes=False),
        out_type=jax.ShapeDtypeStruct((batch * 2, _P), jnp.float32),
        scratch_types=[
            pltpu.VMEM((_P,), jnp.float32),
            pltpu.VMEM((_KS * _T,), jnp.float32),
            pltpu.VMEM((_KS * _PC,), jnp.int32),
        ],
    )(data, t)
    return image.reshape(batch, 2, _NX, _NY)
